# baseline (speedup 1.0000x reference)
import functools

import jax
import jax.numpy as jnp
from jax import lax
from jax.experimental import pallas as pl
from jax.experimental.pallas import tpu as pltpu
from jax.experimental.pallas import tpu_sc as plsc

f32 = jnp.float32
i32 = jnp.int32

N = 10000
F_IN = 3703
H = 500
HP = 512
CW = 128
NCH = HP // CW
CP = 128
E = 160000
NC = 2
NS = 16
BATCH = 128
NB = 79
EPT = NB * BATCH
E_PAD = EPT * NS
NA = 10240
RPT = NA // NS
LAST_RPT = N - 15 * RPT
MB = 400

_MESH = plsc.VectorSubcoreMesh(
    core_axis_name="c", subcore_axis_name="s", num_cores=NC, num_subcores=NS)


def _mm1_body(x_ref, w_ref, b_ref, o0, o1, o2, o3):
    acc = jnp.dot(x_ref[...], w_ref[...], preferred_element_type=f32)
    acc = acc + b_ref[...]
    outs = (o0, o1, o2, o3)
    for c in range(NCH):
        outs[c][...] = acc[:, c * CW:(c + 1) * CW]


def _mm1(X, W1p, b1p):
    return pl.pallas_call(
        _mm1_body,
        grid=(N // MB,),
        in_specs=[
            pl.BlockSpec((MB, F_IN), lambda m: (m, 0)),
            pl.BlockSpec((F_IN, HP), lambda m: (0, 0)),
            pl.BlockSpec((1, HP), lambda m: (0, 0)),
        ],
        out_specs=[pl.BlockSpec((MB, CW), lambda m: (m, 0))] * NCH,
        out_shape=[jax.ShapeDtypeStruct((N, CW), f32)] * NCH,
    )(X, W1p, b1p)


def _mm2_body(x0, x1, x2, x3, w_ref, b_ref, o0, o1, o2, o3):
    xs = (x0, x1, x2, x3)
    acc = jnp.broadcast_to(b_ref[...], (MB, HP)).astype(f32)
    for c in range(NCH):
        acc = acc + jnp.dot(jnp.maximum(xs[c][...], 0.0), w_ref[c],
                            preferred_element_type=f32)
    outs = (o0, o1, o2, o3)
    for c in range(NCH):
        outs[c][...] = acc[:, c * CW:(c + 1) * CW]


def _mm2(hs, W2p, b2p):
    return pl.pallas_call(
        _mm2_body,
        grid=(N // MB,),
        in_specs=[pl.BlockSpec((MB, CW), lambda m: (m, 0))] * NCH + [
            pl.BlockSpec((NCH, CW, HP), lambda m: (0, 0, 0)),
            pl.BlockSpec((1, HP), lambda m: (0, 0)),
        ],
        out_specs=[pl.BlockSpec((MB, CW), lambda m: (m, 0))] * NCH,
        out_shape=[jax.ShapeDtypeStruct((N, CW), f32)] * NCH,
    )(*hs, W2p, b2p)


def _mm3_body(x0, x1, x2, x3, w_ref, b_ref, o_ref):
    xs = (x0, x1, x2, x3)
    acc = jnp.broadcast_to(b_ref[...], (MB, CP)).astype(f32)
    for c in range(NCH):
        acc = acc + jnp.dot(jnp.maximum(xs[c][...], 0.0), w_ref[c],
                            preferred_element_type=f32)
    o_ref[...] = acc


def _mm3(hs, W3p, b3p):
    return pl.pallas_call(
        _mm3_body,
        grid=(N // MB,),
        in_specs=[pl.BlockSpec((MB, CW), lambda m: (m, 0))] * NCH + [
            pl.BlockSpec((NCH, CW, CP), lambda m: (0, 0, 0)),
            pl.BlockSpec((1, CP), lambda m: (0, 0)),
        ],
        out_specs=pl.BlockSpec((MB, CP), lambda m: (m, 0)),
        out_shape=jax.ShapeDtypeStruct((N, CP), f32),
    )(*hs, W3p, b3p)


def _ls_body(xa_ref, xb_ref, o_ref):
    x = xa_ref[...] + xb_ref[...]
    col = lax.broadcasted_iota(i32, x.shape, 1)
    valid = col < 6
    xm = jnp.where(valid, x, -1e30)
    m = jnp.max(xm, axis=1, keepdims=True)
    e = jnp.where(valid, jnp.exp(x - m), 0.0)
    s = jnp.sum(e, axis=1, keepdims=True)
    o_ref[...] = x - m - jnp.log(s)


def _log_softmax(xa, xb):
    return pl.pallas_call(
        _ls_body,
        out_shape=jax.ShapeDtypeStruct((N, CP), f32),
    )(xa, xb)


def _edge_pass(tab, acc, srcs, dsts, ews, src_v, dst_v, ew_v, rows_v,
               isems, gsems, ssems, sid, lo, nb, nvr):
    base0 = sid * EPT + lo * BATCH

    def stage(q, i):
        base = base0 + i * BATCH
        pltpu.async_copy(srcs.at[pl.ds(base, BATCH)], src_v.at[q], isems[q])
        pltpu.async_copy(dsts.at[pl.ds(base, BATCH)], dst_v.at[q], isems[q])
        pltpu.async_copy(ews.at[pl.ds(base, BATCH)], ew_v.at[q], isems[q])

    def wait_stage(q):
        pltpu.make_async_copy(srcs.at[pl.ds(0, BATCH)], src_v.at[q], isems[q]).wait()
        pltpu.make_async_copy(dsts.at[pl.ds(0, BATCH)], dst_v.at[q], isems[q]).wait()
        pltpu.make_async_copy(ews.at[pl.ds(0, BATCH)], ew_v.at[q], isems[q]).wait()

    def gather(q, srow):
        pltpu.async_copy(tab.at[src_v.at[q]], rows_v.at[srow], gsems[srow])

    def wait_gather(srow):
        pltpu.make_async_copy(tab.at[src_v.at[0]], rows_v.at[srow],
                              gsems[srow]).wait()

    def scatter(q, srow):
        pltpu.async_copy(rows_v.at[srow], acc.at[dst_v.at[q]], ssems[srow],
                         add=True)

    def wait_scatter(q, srow):
        pltpu.make_async_copy(rows_v.at[srow], acc.at[dst_v.at[q]],
                              ssems[srow]).wait()

    def scale(srow, q):
        def sgrp(g, _):
            wv = ew_v[q, pl.ds(g * 16, 16)]
            for j in range(16):
                w = jnp.broadcast_to(wv[j:j + 1], (16,))
                r = g * 16 + j
                for v in range(nvr):
                    sl = (srow, r, pl.ds(v * 16, 16))
                    rows_v[sl] = rows_v[sl] * w
            return 0
        lax.fori_loop(0, BATCH // 16, sgrp, 0)

    stage(0, 0)
    stage(1, 1)
    stage(2, 2)
    wait_stage(0)
    gather(0, 0)

    def quad(pq, _):
        for j in range(4):
            i = 4 * pq + j
            srow = j % 2
            q = j

            @pl.when(i < nb)
            def _(i=i, srow=srow, q=q):
                @pl.when(i + 1 < nb)
                def _():
                    wait_stage((q + 1) % 4)

                    @pl.when(i >= 1)
                    def _():
                        wait_scatter((q + 3) % 4, 1 - srow)
                    gather((q + 1) % 4, 1 - srow)
                wait_gather(srow)
                scale(srow, q)
                scatter(q, srow)

                @pl.when(i + 3 < nb)
                def _():
                    stage((q + 3) % 4, i + 3)
        return 0

    lax.fori_loop(0, (nb + 3) // 4, quad, 0)
    wait_scatter((nb - 2) % 4, (nb - 2) % 2)
    wait_scatter((nb - 1) % 4, (nb - 1) % 2)


def _writeback(sid, acc, out):
    myrows = pl.ds(sid * RPT, RPT)

    @pl.when(sid < NS - 1)
    def _():
        pltpu.sync_copy(acc.at[myrows], out.at[myrows])

    @pl.when(sid == NS - 1)
    def _():
        last = pl.ds((NS - 1) * RPT, LAST_RPT)
        pltpu.sync_copy(acc.at[last], out.at[last])


_SPMM_SCRATCH = [
    pltpu.VMEM((4, BATCH), i32),
    pltpu.VMEM((4, BATCH), i32),
    pltpu.VMEM((4, BATCH), f32),
    pltpu.VMEM((2, BATCH, CW), f32),
    pltpu.VMEM_SHARED((NA, CW), f32),
] + [pltpu.SemaphoreType.DMA] * 8


def _spmm_wide_body(t0, t1, t2, t3, srcs, dsts, ews, zeros,
                    o0, o1, o2, o3, src_v, dst_v, ew_v, rows_v, acc, *sems):
    cid = lax.axis_index("c")
    sid = lax.axis_index("s")
    tabs = (t0, t1, t2, t3)
    outs = (o0, o1, o2, o3)
    myrows = pl.ds(sid * RPT, RPT)
    isems, gsems, ssems = sems[0:4], sems[4:6], sems[6:8]

    for t in range(2):
        for core in range(NC):
            c = 2 * core + t

            @pl.when(cid == core)
            def _(tab=tabs[c], out=outs[c]):
                pltpu.sync_copy(zeros.at[myrows], acc.at[myrows])
                plsc.subcore_barrier()
                _edge_pass(tab, acc, srcs, dsts, ews, src_v, dst_v, ew_v,
                           rows_v, isems, gsems, ssems, sid, 0, NB, CW // 16)
                plsc.subcore_barrier()
                _writeback(sid, acc, out)
                plsc.subcore_barrier()


def _spmm_wide(tabs, srcs, dsts, ews, zeros):
    fn = pl.kernel(
        _spmm_wide_body,
        out_type=[jax.ShapeDtypeStruct((N, CW), f32)] * NCH,
        mesh=_MESH,
        scratch_types=list(_SPMM_SCRATCH),
    )
    return fn(*tabs, srcs, dsts, ews, zeros)


_NB0 = 40


def _spmm_small_body(tab, srcs, dsts, ews, zeros,
                     out_a, out_b, src_v, dst_v, ew_v, rows_v, acc, *sems):
    cid = lax.axis_index("c")
    sid = lax.axis_index("s")
    myrows = pl.ds(sid * RPT, RPT)
    isems, gsems, ssems = sems[0:4], sems[4:6], sems[6:8]

    for core, out, lo, nb in ((0, out_a, 0, _NB0), (1, out_b, _NB0, NB - _NB0)):
        @pl.when(cid == core)
        def _(out=out, lo=lo, nb=nb):
            pltpu.sync_copy(zeros.at[myrows], acc.at[myrows])
            plsc.subcore_barrier()
            _edge_pass(tab, acc, srcs, dsts, ews, src_v, dst_v, ew_v,
                       rows_v, isems, gsems, ssems, sid, lo, nb, CP // 16)
            plsc.subcore_barrier()
            _writeback(sid, acc, out)


def _spmm_small(tab, srcs, dsts, ews, zeros):
    fn = pl.kernel(
        _spmm_small_body,
        out_type=[jax.ShapeDtypeStruct((N, CP), f32)] * 2,
        mesh=_MESH,
        scratch_types=list(_SPMM_SCRATCH),
    )
    return fn(tab, srcs, dsts, ews, zeros)


def kernel(X, edge_index, edge_weight, W1, b1, W2, b2, W3, b3):
    order = jnp.argsort(edge_index[0])
    src = edge_index[0][order]
    dst = edge_index[1][order]
    edge_weight = edge_weight[order]
    pad = E_PAD - E
    srcs = jnp.concatenate([src, jnp.zeros((pad,), i32)])
    dsts = jnp.concatenate([dst, jnp.zeros((pad,), i32)])
    ews = jnp.concatenate([edge_weight, jnp.zeros((pad,), f32)])
    zeros = jnp.zeros((NA, CW), f32)

    W1p = jnp.pad(W1, ((0, 0), (0, HP - H)))
    b1p = jnp.pad(b1, (0, HP - H)).reshape(1, HP)
    W2p = jnp.pad(W2, ((0, HP - H), (0, HP - H))).reshape(NCH, CW, HP)
    b2p = jnp.pad(b2, (0, HP - H)).reshape(1, HP)
    W3p = jnp.pad(W3, ((0, HP - H), (0, CP - 6))).reshape(NCH, CW, CP)
    b3p = jnp.pad(b3, (0, CP - 6)).reshape(1, CP)

    s1 = _mm1(X, W1p, b1p)
    h1 = _spmm_wide(s1, srcs, dsts, ews, zeros)
    s2 = _mm2(h1, W2p, b2p)
    h2 = _spmm_wide(s2, srcs, dsts, ews, zeros)
    s3 = _mm3(h2, W3p, b3p)
    o3a, o3b = _spmm_small(s3, srcs, dsts, ews, zeros)
    return _log_softmax(o3a, o3b)[:, :6]


# speedup vs baseline: 1.4507x; 1.4507x over previous
import functools

import jax
import jax.numpy as jnp
from jax import lax
from jax.experimental import pallas as pl
from jax.experimental.pallas import tpu as pltpu
from jax.experimental.pallas import tpu_sc as plsc

f32 = jnp.float32
i32 = jnp.int32

N = 10000
F_IN = 3703
H = 500
HP = 512
CW = 128
NCH = HP // CW
CP = 128
E = 160000
NC = 2
NS = 16
BATCH = 64
NB = 158
EPT = NB * BATCH
E_PAD = EPT * NS
NA = 10240
RPT = NA // NS
LAST_RPT = N - 15 * RPT
MB = 400

_MESH = plsc.VectorSubcoreMesh(
    core_axis_name="c", subcore_axis_name="s", num_cores=NC, num_subcores=NS)


def _mm1_body(x_ref, w_ref, b_ref, o0, o1, o2, o3):
    acc = jnp.dot(x_ref[...], w_ref[...], preferred_element_type=f32)
    acc = acc + b_ref[...]
    outs = (o0, o1, o2, o3)
    for c in range(NCH):
        outs[c][...] = acc[:, c * CW:(c + 1) * CW]


def _mm1(X, W1p, b1p):
    return pl.pallas_call(
        _mm1_body,
        grid=(N // MB,),
        in_specs=[
            pl.BlockSpec((MB, F_IN), lambda m: (m, 0)),
            pl.BlockSpec((F_IN, HP), lambda m: (0, 0)),
            pl.BlockSpec((1, HP), lambda m: (0, 0)),
        ],
        out_specs=[pl.BlockSpec((MB, CW), lambda m: (m, 0))] * NCH,
        out_shape=[jax.ShapeDtypeStruct((N, CW), f32)] * NCH,
    )(X, W1p, b1p)


def _mm2_body(x0, x1, x2, x3, w_ref, b_ref, o0, o1, o2, o3):
    xs = (x0, x1, x2, x3)
    acc = jnp.broadcast_to(b_ref[...], (MB, HP)).astype(f32)
    for c in range(NCH):
        acc = acc + jnp.dot(jnp.maximum(xs[c][...], 0.0), w_ref[c],
                            preferred_element_type=f32)
    outs = (o0, o1, o2, o3)
    for c in range(NCH):
        outs[c][...] = acc[:, c * CW:(c + 1) * CW]


def _mm2(hs, W2p, b2p):
    return pl.pallas_call(
        _mm2_body,
        grid=(N // MB,),
        in_specs=[pl.BlockSpec((MB, CW), lambda m: (m, 0))] * NCH + [
            pl.BlockSpec((NCH, CW, HP), lambda m: (0, 0, 0)),
            pl.BlockSpec((1, HP), lambda m: (0, 0)),
        ],
        out_specs=[pl.BlockSpec((MB, CW), lambda m: (m, 0))] * NCH,
        out_shape=[jax.ShapeDtypeStruct((N, CW), f32)] * NCH,
    )(*hs, W2p, b2p)


def _mm3_body(x0, x1, x2, x3, w_ref, b_ref, o_ref):
    xs = (x0, x1, x2, x3)
    acc = jnp.broadcast_to(b_ref[...], (MB, CP)).astype(f32)
    for c in range(NCH):
        acc = acc + jnp.dot(jnp.maximum(xs[c][...], 0.0), w_ref[c],
                            preferred_element_type=f32)
    o_ref[...] = acc


def _mm3(hs, W3p, b3p):
    return pl.pallas_call(
        _mm3_body,
        grid=(N // MB,),
        in_specs=[pl.BlockSpec((MB, CW), lambda m: (m, 0))] * NCH + [
            pl.BlockSpec((NCH, CW, CP), lambda m: (0, 0, 0)),
            pl.BlockSpec((1, CP), lambda m: (0, 0)),
        ],
        out_specs=pl.BlockSpec((MB, CP), lambda m: (m, 0)),
        out_shape=jax.ShapeDtypeStruct((N, CP), f32),
    )(*hs, W3p, b3p)


def _ls_body(xa_ref, xb_ref, o_ref):
    x = xa_ref[...] + xb_ref[...]
    col = lax.broadcasted_iota(i32, x.shape, 1)
    valid = col < 6
    xm = jnp.where(valid, x, -1e30)
    m = jnp.max(xm, axis=1, keepdims=True)
    e = jnp.where(valid, jnp.exp(x - m), 0.0)
    s = jnp.sum(e, axis=1, keepdims=True)
    o_ref[...] = x - m - jnp.log(s)


def _log_softmax(xa, xb):
    return pl.pallas_call(
        _ls_body,
        out_shape=jax.ShapeDtypeStruct((N, CP), f32),
    )(xa, xb)


def _edge_pass(tab, acc, srcs, dsts, ews, src_v, dst_v, ew_v, rows_v,
               isems, gsems, ssems, sid, lo, nb, nvr):
    base0 = sid * EPT + lo * BATCH

    def stage(i, q4, q8):
        base = base0 + i * BATCH
        pltpu.async_copy(srcs.at[pl.ds(base, BATCH)], src_v.at[q4], isems[q4])
        pltpu.async_copy(dsts.at[pl.ds(base, BATCH)], dst_v.at[q8], isems[q4])
        pltpu.async_copy(ews.at[pl.ds(base, BATCH)], ew_v.at[q4], isems[q4])

    def wait_stage(q4, q8):
        pltpu.make_async_copy(srcs.at[pl.ds(0, BATCH)], src_v.at[q4],
                              isems[q4]).wait()
        pltpu.make_async_copy(dsts.at[pl.ds(0, BATCH)], dst_v.at[q8],
                              isems[q4]).wait()
        pltpu.make_async_copy(ews.at[pl.ds(0, BATCH)], ew_v.at[q4],
                              isems[q4]).wait()

    def gather(q4):
        pltpu.async_copy(tab.at[src_v.at[q4]], rows_v.at[q4], gsems[q4])

    def wait_gather(q4):
        pltpu.make_async_copy(tab.at[src_v.at[0]], rows_v.at[q4],
                              gsems[q4]).wait()

    def scatter(q8, q4):
        pltpu.async_copy(rows_v.at[q4], acc.at[dst_v.at[q8]], ssems[q4],
                         add=True)

    def wait_scatter(q8, q4):
        pltpu.make_async_copy(rows_v.at[q4], acc.at[dst_v.at[q8]],
                              ssems[q4]).wait()

    def scale(q4, nvr=nvr):
        def sgrp(g, _):
            wv = ew_v[q4, pl.ds(g * 16, 16)]
            for j in range(16):
                w = jnp.broadcast_to(wv[j:j + 1], (16,))
                r = g * 16 + j
                for v in range(nvr):
                    sl = (q4, r, pl.ds(v * 16, 16))
                    rows_v[sl] = rows_v[sl] * w
            return 0
        lax.fori_loop(0, BATCH // 16, sgrp, 0)

    for k in range(4):
        stage(k, k, k)
    wait_stage(0, 0)
    gather(0)
    wait_stage(1, 1)
    gather(1)

    def octet(po, _):
        for j in range(8):
            i = 8 * po + j
            q4 = j % 4
            q8 = j

            @pl.when(i < nb)
            def _(i=i, q4=q4, q8=q8):
                @pl.when(i + 2 < nb)
                def _():
                    wait_stage((q4 + 2) % 4, (q8 + 2) % 8)

                    @pl.when(i >= 2)
                    def _():
                        wait_scatter((q8 + 6) % 8, (q4 + 2) % 4)
                    gather((q4 + 2) % 4)
                wait_gather(q4)
                scale(q4)
                scatter(q8, q4)

                @pl.when(i + 4 < nb)
                def _():
                    stage(i + 4, q4, (q8 + 4) % 8)
        return 0

    lax.fori_loop(0, (nb + 7) // 8, octet, 0)
    for k in (4, 3, 2, 1):
        wait_scatter((nb - k) % 8, (nb - k) % 4)


def _writeback(sid, acc, out):
    myrows = pl.ds(sid * RPT, RPT)

    @pl.when(sid < NS - 1)
    def _():
        pltpu.sync_copy(acc.at[myrows], out.at[myrows])

    @pl.when(sid == NS - 1)
    def _():
        last = pl.ds((NS - 1) * RPT, LAST_RPT)
        pltpu.sync_copy(acc.at[last], out.at[last])


_SPMM_SCRATCH = [
    pltpu.VMEM((4, BATCH), i32),
    pltpu.VMEM((8, BATCH), i32),
    pltpu.VMEM((4, BATCH), f32),
    pltpu.VMEM((4, BATCH, CW), f32),
    pltpu.VMEM_SHARED((NA, CW), f32),
] + [pltpu.SemaphoreType.DMA] * 12


def _spmm_wide_body(t0, t1, t2, t3, srcs, dsts, ews, zeros,
                    o0, o1, o2, o3, src_v, dst_v, ew_v, rows_v, acc, *sems):
    cid = lax.axis_index("c")
    sid = lax.axis_index("s")
    tabs = (t0, t1, t2, t3)
    outs = (o0, o1, o2, o3)
    myrows = pl.ds(sid * RPT, RPT)
    isems, gsems, ssems = sems[0:4], sems[4:8], sems[8:12]

    for t in range(2):
        for core in range(NC):
            c = 2 * core + t

            @pl.when(cid == core)
            def _(tab=tabs[c], out=outs[c]):
                pltpu.sync_copy(zeros.at[myrows], acc.at[myrows])
                plsc.subcore_barrier()
                _edge_pass(tab, acc, srcs, dsts, ews, src_v, dst_v, ew_v,
                           rows_v, isems, gsems, ssems, sid, 0, NB, CW // 16)
                plsc.subcore_barrier()
                _writeback(sid, acc, out)
                plsc.subcore_barrier()


def _spmm_wide(tabs, srcs, dsts, ews, zeros):
    fn = pl.kernel(
        _spmm_wide_body,
        out_type=[jax.ShapeDtypeStruct((N, CW), f32)] * NCH,
        mesh=_MESH,
        scratch_types=list(_SPMM_SCRATCH),
    )
    return fn(*tabs, srcs, dsts, ews, zeros)


_NB0 = 80


def _spmm_small_body(tab, srcs, dsts, ews, zeros,
                     out_a, out_b, src_v, dst_v, ew_v, rows_v, acc, *sems):
    cid = lax.axis_index("c")
    sid = lax.axis_index("s")
    myrows = pl.ds(sid * RPT, RPT)
    isems, gsems, ssems = sems[0:4], sems[4:8], sems[8:12]

    for core, out, lo, nb in ((0, out_a, 0, _NB0), (1, out_b, _NB0, NB - _NB0)):
        @pl.when(cid == core)
        def _(out=out, lo=lo, nb=nb):
            pltpu.sync_copy(zeros.at[myrows], acc.at[myrows])
            plsc.subcore_barrier()
            _edge_pass(tab, acc, srcs, dsts, ews, src_v, dst_v, ew_v,
                       rows_v, isems, gsems, ssems, sid, lo, nb, CP // 16)
            plsc.subcore_barrier()
            _writeback(sid, acc, out)


def _spmm_small(tab, srcs, dsts, ews, zeros):
    fn = pl.kernel(
        _spmm_small_body,
        out_type=[jax.ShapeDtypeStruct((N, CP), f32)] * 2,
        mesh=_MESH,
        scratch_types=list(_SPMM_SCRATCH),
    )
    return fn(tab, srcs, dsts, ews, zeros)


def kernel(X, edge_index, edge_weight, W1, b1, W2, b2, W3, b3):
    src = edge_index[0]
    dst = edge_index[1]
    pad = E_PAD - E
    srcs = jnp.concatenate([src, jnp.zeros((pad,), i32)])
    dsts = jnp.concatenate([dst, jnp.zeros((pad,), i32)])
    ews = jnp.concatenate([edge_weight, jnp.zeros((pad,), f32)])
    zeros = jnp.zeros((NA, CW), f32)

    W1p = jnp.pad(W1, ((0, 0), (0, HP - H)))
    b1p = jnp.pad(b1, (0, HP - H)).reshape(1, HP)
    W2p = jnp.pad(W2, ((0, HP - H), (0, HP - H))).reshape(NCH, CW, HP)
    b2p = jnp.pad(b2, (0, HP - H)).reshape(1, HP)
    W3p = jnp.pad(W3, ((0, HP - H), (0, CP - 6))).reshape(NCH, CW, CP)
    b3p = jnp.pad(b3, (0, CP - 6)).reshape(1, CP)

    s1 = _mm1(X, W1p, b1p)
    h1 = _spmm_wide(s1, srcs, dsts, ews, zeros)
    s2 = _mm2(h1, W2p, b2p)
    h2 = _spmm_wide(s2, srcs, dsts, ews, zeros)
    s3 = _mm3(h2, W3p, b3p)
    o3a, o3b = _spmm_small(s3, srcs, dsts, ews, zeros)
    return _log_softmax(o3a, o3b)[:, :6]
